# Initial kernel scaffold; baseline (speedup 1.0000x reference)
import functools

import jax
import jax.numpy as jnp
from jax import lax
from jax.experimental import pallas as pl
from jax.experimental.pallas import tpu as pltpu
from jax.experimental.pallas import tpu_sc as plsc

_V = 10000
_E = 320000
_HID = 128
_OUT = 128

_NC = 2
_NS = 16
_NW = _NC * _NS
_EPS = _E // _NW
_CS = 80
_NCH = _EPS // _CS
_GRP = 25
_NGRP = _NCH // _GRP
_RPS = _V // _NS
_ZR = 125
_FS = _HID // 16


def _sc_spmm_body(src0, dst0, val0, src1, dst1, val1, w0, w1, out,
                  acc, sidx, didx, vals, rows, zbuf):
    c = lax.axis_index("c")
    s = lax.axis_index("s")
    wid = c * _NS + s

    def _zrow(r, carry):
        for f in range(_FS):
            zbuf[r, pl.ds(f * 16, 16)] = jnp.zeros((16,), jnp.float32)
        return carry
    lax.fori_loop(0, _ZR, _zrow, 0)
    for j in range(_RPS // _ZR):
        pltpu.sync_copy(zbuf, acc.at[pl.ds(s * _RPS + j * _ZR, _ZR)])
    plsc.subcore_barrier()

    for (srcr, dstr, valr, wr) in ((src0, dst0, val0, w0),
                                   (src1, dst1, val1, w1)):
        def _group(g, carry):
            pltpu.sync_copy(srcr.at[wid, pl.ds(g * _GRP, _GRP)], sidx)
            pltpu.sync_copy(dstr.at[wid, pl.ds(g * _GRP, _GRP)], didx)
            pltpu.sync_copy(valr.at[wid, pl.ds(g * _GRP, _GRP)], vals)

            def _chunk(jj, carry2):
                pltpu.sync_copy(wr.at[sidx.at[jj]], rows)

                def _edge(i, carry3):
                    vi = plsc.load_gather(
                        vals, [jnp.full((16,), jj, jnp.int32),
                               jnp.full((16,), i, jnp.int32)])
                    for f in range(_FS):
                        sl = pl.ds(f * 16, 16)
                        rows[i, sl] = rows[i, sl] * vi
                    return carry3
                lax.fori_loop(0, _CS, _edge, 0)

                pltpu.sync_copy(rows, acc.at[didx.at[jj]], add=True)
                return carry2
            lax.fori_loop(0, _GRP, _chunk, 0)
            return carry
        lax.fori_loop(0, _NGRP, _group, 0)

    plsc.subcore_barrier()
    pltpu.sync_copy(acc.at[pl.ds(s * _RPS, _RPS)],
                    out.at[c, pl.ds(s * _RPS, _RPS)])


_sc_spmm = functools.partial(
    pl.kernel,
    out_type=jax.ShapeDtypeStruct((_NC, _V, _HID), jnp.float32),
    mesh=plsc.VectorSubcoreMesh(core_axis_name="c", subcore_axis_name="s"),
    scratch_types=[
        pltpu.VMEM_SHARED((_V, _HID), jnp.float32),
        pltpu.VMEM((_GRP, _CS), jnp.int32),
        pltpu.VMEM((_GRP, _CS), jnp.int32),
        pltpu.VMEM((_GRP, _CS), jnp.float32),
        pltpu.VMEM((_CS, _HID), jnp.float32),
        pltpu.VMEM((_ZR, _HID), jnp.float32),
    ],
)(_sc_spmm_body)


_BD = 256
_KT = 2000
_NK = _V // _KT


def _mm_body(x_ref, h_ref, w_ref, b_ref, o_ref, acc_ref):
    k = pl.program_id(0)

    @pl.when(k == 0)
    def _init():
        acc_ref[...] = jnp.zeros_like(acc_ref)

    hs = h_ref[0] + h_ref[1]
    acc_ref[...] += jnp.dot(x_ref[...], hs, preferred_element_type=jnp.float32)

    @pl.when(k == pl.num_programs(0) - 1)
    def _fin():
        o_ref[...] = lax.dot_general(
            acc_ref[...], w_ref[...], (((1,), (1,)), ((), ())),
            preferred_element_type=jnp.float32) + b_ref[...]


_mm = pl.pallas_call(
    _mm_body,
    grid=(_NK,),
    in_specs=[
        pl.BlockSpec((_BD, _KT), lambda k: (0, k)),
        pl.BlockSpec((_NC, _KT, _HID), lambda k: (0, k, 0)),
        pl.BlockSpec((_OUT, _HID), lambda k: (0, 0)),
        pl.BlockSpec((1, _OUT), lambda k: (0, 0)),
    ],
    out_specs=pl.BlockSpec((_BD, _OUT), lambda k: (0, 0)),
    out_shape=jax.ShapeDtypeStruct((_BD, _OUT), jnp.float32),
    scratch_shapes=[pltpu.VMEM((_BD, _OUT), jnp.float32)],
    compiler_params=pltpu.CompilerParams(
        dimension_semantics=("arbitrary",)),
)


def kernel(adj0_indices, adj0_values, adj1_indices, adj1_values, X_dv,
           W0, W1, fc_w, fc_b):
    B, D, V = X_dv.shape

    def _split(idx, vals):
        idx = idx.astype(jnp.int32)
        src = idx[1].reshape(_NW, _NCH, _CS)
        dst = idx[0].reshape(_NW, _NCH, _CS)
        val = vals.reshape(_NW, _NCH, _CS)
        return src, dst, val

    s0, d0, v0 = _split(adj0_indices, adj0_values)
    s1, d1, v1 = _split(adj1_indices, adj1_values)

    hpart = _sc_spmm(s0, d0, v0, s1, d1, v1, W0, W1)

    x2 = X_dv.reshape(B * D, V)
    out2 = _mm(x2, hpart, fc_w, fc_b.reshape(1, _OUT))
    return out2.reshape(B, D, _OUT)


# speedup vs baseline: 4.9212x; 4.9212x over previous
import functools

import jax
import jax.numpy as jnp
from jax import lax
from jax.experimental import pallas as pl
from jax.experimental.pallas import tpu as pltpu
from jax.experimental.pallas import tpu_sc as plsc

_V = 10000
_E = 320000
_HID = 128
_OUT = 128

_NC = 2
_NS = 16
_NW = _NC * _NS
_EPS = _E // _NW
_CS = 80
_NCH = _EPS // _CS
_GRP = 25
_NGRP = _NCH // _GRP
_RPS = _V // _NS
_ZR = 125
_FS = _HID // 16


def _sc_spmm_body(src0, dst0, val0, src1, dst1, val1, w0, w1, out,
                  acc, sidx, didx, vals, rows, zbuf):
    c = lax.axis_index("c")
    s = lax.axis_index("s")
    wid = c * _NS + s

    def _zrow(r, carry):
        for f in range(_FS):
            zbuf[r, pl.ds(f * 16, 16)] = jnp.zeros((16,), jnp.float32)
        return carry
    lax.fori_loop(0, _ZR, _zrow, 0)
    for j in range(_RPS // _ZR):
        pltpu.sync_copy(zbuf, acc.at[pl.ds(s * _RPS + j * _ZR, _ZR)])
    plsc.subcore_barrier()

    for (srcr, dstr, valr, wr) in ((src0, dst0, val0, w0),
                                   (src1, dst1, val1, w1)):
        def _group(g, carry):
            pltpu.sync_copy(srcr.at[wid, pl.ds(g * _GRP, _GRP)], sidx)
            pltpu.sync_copy(dstr.at[wid, pl.ds(g * _GRP, _GRP)], didx)
            pltpu.sync_copy(valr.at[wid, pl.ds(g * _GRP, _GRP)], vals)

            def _chunk(jj, carry2):
                pltpu.sync_copy(wr.at[sidx.at[jj]], rows)

                def _edge(i, carry3):
                    vi = plsc.load_gather(
                        vals, [jnp.full((16,), jj, jnp.int32),
                               jnp.full((16,), i, jnp.int32)])
                    for f in range(_FS):
                        sl = pl.ds(f * 16, 16)
                        rows[i, sl] = rows[i, sl] * vi
                    return carry3
                lax.fori_loop(0, _CS, _edge, 0)

                pltpu.sync_copy(rows, acc.at[didx.at[jj]], add=True)
                return carry2
            lax.fori_loop(0, _GRP, _chunk, 0)
            return carry
        lax.fori_loop(0, _NGRP, _group, 0)

    plsc.subcore_barrier()
    pltpu.sync_copy(acc.at[pl.ds(s * _RPS, _RPS)],
                    out.at[c, pl.ds(s * _RPS, _RPS)])


@functools.lru_cache(maxsize=None)
def _make_sc_spmm():
  return functools.partial(
    pl.kernel,
    out_type=jax.ShapeDtypeStruct((_NC, _V, _HID), jnp.float32),
    mesh=plsc.VectorSubcoreMesh(core_axis_name="c", subcore_axis_name="s",
                                num_cores=_NC, num_subcores=_NS),
    scratch_types=[
        pltpu.VMEM_SHARED((_V, _HID), jnp.float32),
        pltpu.VMEM((_GRP, _CS), jnp.int32),
        pltpu.VMEM((_GRP, _CS), jnp.int32),
        pltpu.VMEM((_GRP, _CS), jnp.float32),
        pltpu.VMEM((_CS, _HID), jnp.float32),
        pltpu.VMEM((_ZR, _HID), jnp.float32),
    ],
    compiler_params=pltpu.CompilerParams(use_tc_tiling_on_sc=False,
                                         needs_layout_passes=False),
  )(_sc_spmm_body)


_BD = 256


def _mm_body(x_ref, h_ref, w_ref, b_ref, o_ref):
    hs = h_ref[0] + h_ref[1]
    acc = jnp.dot(x_ref[...], hs, preferred_element_type=jnp.float32)
    o_ref[...] = lax.dot_general(
        acc, w_ref[...], (((1,), (1,)), ((), ())),
        preferred_element_type=jnp.float32) + b_ref[...]


_mm = pl.pallas_call(
    _mm_body,
    out_shape=jax.ShapeDtypeStruct((_BD, _OUT), jnp.float32),
)


def kernel(adj0_indices, adj0_values, adj1_indices, adj1_values, X_dv,
           W0, W1, fc_w, fc_b):
    B, D, V = X_dv.shape

    def _split(idx, vals):
        idx = idx.astype(jnp.int32)
        src = idx[1].reshape(_NW, _NCH, _CS)
        dst = idx[0].reshape(_NW, _NCH, _CS)
        val = vals.reshape(_NW, _NCH, _CS)
        return src, dst, val

    s0, d0, v0 = _split(adj0_indices, adj0_values)
    s1, d1, v1 = _split(adj1_indices, adj1_values)

    hpart = _make_sc_spmm()(s0, d0, v0, s1, d1, v1, W0, W1)

    x2 = X_dv.reshape(B * D, V)
    out2 = _mm(x2, hpart, fc_w, fc_b.reshape(1, _OUT))
    return out2.reshape(B, D, _OUT)


# speedup vs baseline: 6.4439x; 1.3094x over previous
import functools

import jax
import jax.numpy as jnp
from jax import lax
from jax.experimental import pallas as pl
from jax.experimental.pallas import tpu as pltpu
from jax.experimental.pallas import tpu_sc as plsc

_V = 10000
_E = 320000
_HID = 128
_OUT = 128

_NC = 2
_NS = 16
_NW = _NC * _NS
_EPS = _E // _NW
_CS = 40
_NH = 2
_NCHH = _EPS // _NH // _CS
_RPS = _V // _NS
_ZR = 25
_FS = _HID // 16


def _sc_spmm_body(src0, dst0, val0, src1, dst1, val1, w0, w1, out,
                  acc, sidx, didx, vals, rows, zbuf, gsem, ssem):
    c = lax.axis_index("c")
    s = lax.axis_index("s")
    wid = c * _NS + s

    def _zrow(r, carry):
        for f in range(_FS):
            zbuf[r, pl.ds(f * 16, 16)] = jnp.zeros((16,), jnp.float32)
        return carry
    lax.fori_loop(0, _ZR, _zrow, 0)
    for j in range(_RPS // _ZR):
        pltpu.sync_copy(zbuf, acc.at[pl.ds(s * _RPS + j * _ZR, _ZR)])
    plsc.subcore_barrier()

    for (srcr, dstr, valr, wr) in ((src0, dst0, val0, w0),
                                   (src1, dst1, val1, w1)):
      for h in range(_NH):
        pltpu.sync_copy(srcr.at[wid, h], sidx)
        pltpu.sync_copy(dstr.at[wid, h], didx)
        pltpu.sync_copy(valr.at[wid, h], vals)

        def g_issue(ch, b):
            pltpu.async_copy(wr.at[sidx.at[ch]], rows.at[b], gsem)

        def g_wait(ch, b):
            pltpu.make_async_copy(
                wr.at[sidx.at[ch]], rows.at[b], gsem).wait()

        def s_issue(ch, b):
            pltpu.async_copy(rows.at[b], acc.at[didx.at[ch]], ssem,
                             add=True)

        def s_wait(b):
            pltpu.make_async_copy(
                rows.at[b], acc.at[didx.at[0]], ssem).wait()

        def scale(ch, b):
            def _edge(i, carry):
                vi = plsc.load_gather(
                    vals, [jnp.full((16,), ch, jnp.int32),
                           jnp.full((16,), i, jnp.int32)])
                for f in range(_FS):
                    sl = pl.ds(f * 16, 16)
                    rows[b, i, sl] = rows[b, i, sl] * vi
                return carry
            lax.fori_loop(0, _CS, _edge, 0)

        g_issue(0, 0)
        g_wait(0, 0)
        g_issue(1, 1)
        scale(0, 0)
        s_issue(0, 0)

        def _pair(t, carry):
            for (off, b) in ((1, 1), (2, 0)):
                ch = 2 * t + off
                g_wait(ch, b)
                s_wait(1 - b)
                g_issue(ch + 1, 1 - b)
                scale(ch, b)
                s_issue(ch, b)
            return carry
        lax.fori_loop(0, (_NCHH - 3) // 2, _pair, 0)

        g_wait(_NCHH - 2, 1)
        s_wait(0)
        g_issue(_NCHH - 1, 0)
        scale(_NCHH - 2, 1)
        s_issue(_NCHH - 2, 1)
        g_wait(_NCHH - 1, 0)
        s_wait(1)
        scale(_NCHH - 1, 0)
        s_issue(_NCHH - 1, 0)
        s_wait(0)

    plsc.subcore_barrier()
    pltpu.sync_copy(acc.at[pl.ds(s * _RPS, _RPS)],
                    out.at[c, pl.ds(s * _RPS, _RPS)])


@functools.lru_cache(maxsize=None)
def _make_sc_spmm():
  return functools.partial(
    pl.kernel,
    out_type=jax.ShapeDtypeStruct((_NC, _V, _HID), jnp.float32),
    mesh=plsc.VectorSubcoreMesh(core_axis_name="c", subcore_axis_name="s",
                                num_cores=_NC, num_subcores=_NS),
    scratch_types=[
        pltpu.VMEM_SHARED((_V, _HID), jnp.float32),
        pltpu.VMEM((_NCHH, _CS), jnp.int32),
        pltpu.VMEM((_NCHH, _CS), jnp.int32),
        pltpu.VMEM((_NCHH, _CS), jnp.float32),
        pltpu.VMEM((2, _CS, _HID), jnp.float32),
        pltpu.VMEM((_ZR, _HID), jnp.float32),
        pltpu.SemaphoreType.DMA,
        pltpu.SemaphoreType.DMA,
    ],
    compiler_params=pltpu.CompilerParams(use_tc_tiling_on_sc=False,
                                         needs_layout_passes=False),
  )(_sc_spmm_body)


_BD = 256


def _mm_body(x_ref, h_ref, w_ref, b_ref, o_ref):
    hs = h_ref[0] + h_ref[1]
    acc = jnp.dot(x_ref[...], hs, preferred_element_type=jnp.float32)
    o_ref[...] = lax.dot_general(
        acc, w_ref[...], (((1,), (1,)), ((), ())),
        preferred_element_type=jnp.float32) + b_ref[...]


_mm = pl.pallas_call(
    _mm_body,
    out_shape=jax.ShapeDtypeStruct((_BD, _OUT), jnp.float32),
)


def kernel(adj0_indices, adj0_values, adj1_indices, adj1_values, X_dv,
           W0, W1, fc_w, fc_b):
    B, D, V = X_dv.shape

    def _split(idx, vals):
        idx = idx.astype(jnp.int32)
        src = idx[1].reshape(_NW, _NH, _NCHH, _CS)
        dst = idx[0].reshape(_NW, _NH, _NCHH, _CS)
        val = vals.reshape(_NW, _NH, _NCHH, _CS)
        return src, dst, val

    s0, d0, v0 = _split(adj0_indices, adj0_values)
    s1, d1, v1 = _split(adj1_indices, adj1_values)

    hpart = _make_sc_spmm()(s0, d0, v0, s1, d1, v1, W0, W1)

    x2 = X_dv.reshape(B * D, V)
    out2 = _mm(x2, hpart, fc_w, fc_b.reshape(1, _OUT))
    return out2.reshape(B, D, _OUT)
